# baseline (speedup 1.0000x reference)
import jax
import jax.numpy as jnp
from jax.experimental import pallas as pl
from jax.experimental.pallas import tpu as pltpu


def _body(sidx_ref, ts_ref, ss_ref, x_ref, temb_ref, semb_ref, o_ref, pos_ref):
    t = pl.program_id(0)
    b = pl.program_id(1)

    @pl.when(b == 0)
    def _():
        idx = sidx_ref[0] + t
        trow = temb_ref[pl.ds(idx, 1), :]
        pos_ref[...] = ts_ref[0] * trow + ss_ref[0] * semb_ref[...]

    o_ref[0, 0] = x_ref[0, 0] + pos_ref[...]


def kernel(x, temporal_embed, spatial_embed, temporal_scale, spatial_scale, start_idx):
    B, T, S, D = x.shape
    sidx = jnp.asarray(start_idx, jnp.int32).reshape(1)
    smem = pl.BlockSpec(memory_space=pltpu.SMEM)
    grid = (T, B)
    return pl.pallas_call(
        _body,
        grid=grid,
        in_specs=[
            smem,
            smem,
            smem,
            pl.BlockSpec((1, 1, S, D), lambda t, b: (b, t, 0, 0)),
            pl.BlockSpec((temporal_embed.shape[0], D), lambda t, b: (0, 0)),
            pl.BlockSpec((S, D), lambda t, b: (0, 0)),
        ],
        out_specs=pl.BlockSpec((1, 1, S, D), lambda t, b: (b, t, 0, 0)),
        out_shape=jax.ShapeDtypeStruct((B, T, S, D), x.dtype),
        scratch_shapes=[pltpu.VMEM((S, D), jnp.float32)],
    )(sidx, temporal_scale, spatial_scale, x, temporal_embed, spatial_embed)


# speedup vs baseline: 1.0369x; 1.0369x over previous
import jax
import jax.numpy as jnp
from jax.experimental import pallas as pl
from jax.experimental.pallas import tpu as pltpu


def _body(sidx_ref, ts_ref, ss_ref, x_ref, temb_ref, semb_ref, o_ref, pos_ref):
    t = pl.program_id(0)
    b = pl.program_id(1)

    @pl.when(b == 0)
    def _():
        idx = sidx_ref[0] + t
        trow = temb_ref[pl.ds(idx, 1), :]
        pos_ref[...] = ts_ref[0] * trow + ss_ref[0] * semb_ref[...]

    o_ref[...] = x_ref[...] + pos_ref[...][None, None]


def kernel(x, temporal_embed, spatial_embed, temporal_scale, spatial_scale, start_idx):
    B, T, S, D = x.shape
    sidx = jnp.asarray(start_idx, jnp.int32).reshape(1)
    smem = pl.BlockSpec(memory_space=pltpu.SMEM)
    BB = 2
    grid = (T, B // BB)
    return pl.pallas_call(
        _body,
        grid=grid,
        in_specs=[
            smem,
            smem,
            smem,
            pl.BlockSpec((BB, 1, S, D), lambda t, b: (b, t, 0, 0)),
            pl.BlockSpec((temporal_embed.shape[0], D), lambda t, b: (0, 0)),
            pl.BlockSpec((S, D), lambda t, b: (0, 0)),
        ],
        out_specs=pl.BlockSpec((BB, 1, S, D), lambda t, b: (b, t, 0, 0)),
        out_shape=jax.ShapeDtypeStruct((B, T, S, D), x.dtype),
        scratch_shapes=[pltpu.VMEM((S, D), jnp.float32)],
    )(sidx, temporal_scale, spatial_scale, x, temporal_embed, spatial_embed)


# speedup vs baseline: 1.0485x; 1.0113x over previous
import jax
import jax.numpy as jnp
from jax.experimental import pallas as pl
from jax.experimental.pallas import tpu as pltpu


def _body(sidx_ref, ts_ref, ss_ref, x_ref, temb_ref, semb_ref, o_ref, pos_ref):
    t = pl.program_id(0)
    b = pl.program_id(1)

    @pl.when(b == 0)
    def _():
        idx = sidx_ref[0] + t
        trow = temb_ref[pl.ds(idx, 1), :]
        pos_ref[...] = ts_ref[0] * trow + ss_ref[0] * semb_ref[...]

    o_ref[...] = x_ref[...] + pos_ref[...][None, None]


def kernel(x, temporal_embed, spatial_embed, temporal_scale, spatial_scale, start_idx):
    B, T, S, D = x.shape
    sidx = jnp.asarray(start_idx, jnp.int32).reshape(1)
    smem = pl.BlockSpec(memory_space=pltpu.SMEM)
    BB = 4
    grid = (T, B // BB)
    return pl.pallas_call(
        _body,
        grid=grid,
        in_specs=[
            smem,
            smem,
            smem,
            pl.BlockSpec((BB, 1, S, D), lambda t, b: (b, t, 0, 0)),
            pl.BlockSpec((temporal_embed.shape[0], D), lambda t, b: (0, 0)),
            pl.BlockSpec((S, D), lambda t, b: (0, 0)),
        ],
        out_specs=pl.BlockSpec((BB, 1, S, D), lambda t, b: (b, t, 0, 0)),
        out_shape=jax.ShapeDtypeStruct((B, T, S, D), x.dtype),
        scratch_shapes=[pltpu.VMEM((S, D), jnp.float32)],
    )(sidx, temporal_scale, spatial_scale, x, temporal_embed, spatial_embed)
